# baseline (speedup 1.0000x reference)
import functools

import jax
import jax.numpy as jnp
from jax import lax
from jax.experimental import pallas as pl
from jax.experimental.pallas import tpu as pltpu
from jax.experimental.pallas import tpu_sc as plsc

N_NODES = 10000
KNN_K = 32
_NP = 10240
_R = 128


def _knn_body(rows_ref, cols_ref, sqi_ref, sqj_ref, out_ref, *, n_valid, k):
    i = pl.program_id(0)
    rows = rows_ref[...]
    cols = cols_ref[...]
    npad = cols.shape[1]
    r = rows.shape[0]
    sqi = sqi_ref[...][:, :1]
    sqj = sqj_ref[...][:1, :]
    mm = jnp.dot(rows, cols, preferred_element_type=jnp.float32)
    s = (sqi + sqj) - 2.0 * mm
    col_iota = lax.broadcasted_iota(jnp.int32, (r, npad), 1)
    row_idx = i * r + lax.broadcasted_iota(jnp.int32, (r, npad), 0)
    s = s + jnp.where(col_iota == row_idx, jnp.float32(1e10), jnp.float32(0.0))
    s = jnp.where(col_iota >= n_valid, jnp.float32(jnp.inf), s)
    picks = []
    for _ in range(k):
        m = jnp.min(s, axis=1, keepdims=True)
        idx = jnp.min(jnp.where(s <= m, col_iota, npad), axis=1, keepdims=True)
        picks.append(idx)
        s = jnp.where(col_iota == idx, jnp.float32(jnp.inf), s)
    out_ref[...] = jnp.concatenate(picks, axis=1)


def _knn_pallas(x, k=KNN_K):
    n, f = x.shape
    fp = max(8, ((f + 7) // 8) * 8)
    xp = jnp.zeros((_NP, fp), jnp.float32).at[:n, :f].set(x)
    cols = xp.T
    sq = jnp.sum(x * x, axis=1)
    sqp = jnp.zeros((_NP,), jnp.float32).at[:n].set(sq)
    sqi_in = jnp.tile(sqp[:, None], (1, 8))
    sqj_in = jnp.tile(sqp[None, :], (8, 1))
    grid = (_NP // _R,)
    out = pl.pallas_call(
        functools.partial(_knn_body, n_valid=n, k=k),
        grid=grid,
        in_specs=[
            pl.BlockSpec((_R, fp), lambda i: (i, 0)),
            pl.BlockSpec((fp, _NP), lambda i: (0, 0)),
            pl.BlockSpec((_R, 8), lambda i: (i, 0)),
            pl.BlockSpec((8, _NP), lambda i: (0, 0)),
        ],
        out_specs=pl.BlockSpec((_R, k), lambda i: (i, 0)),
        out_shape=jax.ShapeDtypeStruct((_NP, k), jnp.int32),
    )(xp, cols, sqi_in, sqj_in)
    return out[:n]


def _mlp_apply(layers, h):
    for l in layers:
        h = h @ l["W"] + l["b"]
        h = jax.nn.relu(h)
        m = h.mean(0)
        v = h.var(0)
        h = (h - m) / jnp.sqrt(v + 1e-5) * l["g"] + l["be"]
    return h


def _dyn_edge_conv(layers, x, k):
    idx = _knn_pallas(x, k)
    n = x.shape[0]
    xi = jnp.broadcast_to(x[:, None, :], (n, k, x.shape[1]))
    xj = x[idx]
    h = jnp.concatenate([xi, xj - xi], axis=-1).reshape(n * k, -1)
    h = _mlp_apply(layers, h)
    return h.reshape(n, k, -1).max(axis=1)


_NW = 32
_NCH = 80
_NPAD_SC = 10112


def _make_sc_hop(d):
    mesh = plsc.VectorSubcoreMesh(core_axis_name="c", subcore_axis_name="s")
    rpt = _NPAD_SC // 16

    @functools.partial(
        pl.kernel, mesh=mesh,
        out_type=jax.ShapeDtypeStruct((2, _NPAD_SC, d), jnp.float32),
        scratch_types=[
            pltpu.VMEM((_NCH, 128), jnp.int32),
            pltpu.VMEM((_NCH, 128), jnp.int32),
            pltpu.VMEM((128, d), jnp.float32),
            pltpu.VMEM_SHARED((_NPAD_SC, d), jnp.float32),
            pltpu.SemaphoreType.DMA,
        ],
    )
    def hop(u_hbm, srcw_hbm, dstw_hbm, zeros_hbm, out_hbm,
            src_v, dst_v, rows_v, accum, sem):
        c = lax.axis_index("c")
        s = lax.axis_index("s")
        w = s * 2 + c
        pltpu.sync_copy(zeros_hbm.at[pl.ds(s * rpt, rpt)],
                        accum.at[pl.ds(s * rpt, rpt)])
        pltpu.sync_copy(srcw_hbm.at[w], src_v)
        pltpu.sync_copy(dstw_hbm.at[w], dst_v)
        plsc.subcore_barrier()

        def body(j, carry):
            pltpu.async_copy(u_hbm.at[src_v.at[j]], rows_v, sem).wait()
            pltpu.sync_copy(rows_v, accum.at[dst_v.at[j]], add=True)
            return carry

        lax.fori_loop(0, _NCH, body, 0)
        plsc.subcore_barrier()
        pltpu.sync_copy(accum.at[pl.ds(s * rpt, rpt)],
                        out_hbm.at[c, pl.ds(s * rpt, rpt)])

    return hop


_sc_hop_128 = _make_sc_hop(128)


def _sc_segment_sum(u_pad, srcw, dstw, zeros_pad, d):
    parts = _sc_hop_128(u_pad, srcw, dstw, zeros_pad)
    return parts[0] + parts[1]


def _pad_edges(src, dst, n):
    e_cap = _NW * _NCH * 128
    e = src.shape[0]
    srcp = jnp.full((e_cap,), n, jnp.int32).at[:e].set(src)
    dstp = jnp.full((e_cap,), n, jnp.int32).at[:e].set(dst)
    return (srcp.reshape(_NW, _NCH, 128), dstp.reshape(_NW, _NCH, 128))


def _tag_conv_sc(p, x, srcw, dstw, dis, n, hops=3):
    f = x.shape[1]
    d = 128
    zeros_pad = jnp.zeros((_NPAD_SC, d), jnp.float32)
    dis_col = dis[:, None]
    xs = [x]
    h = x
    for _ in range(hops):
        u = jnp.zeros((_NPAD_SC, d), jnp.float32).at[:n, :f].set(h * dis_col)
        s = _sc_segment_sum(u, srcw, dstw, zeros_pad, d)
        h = s[:n, :f] * dis_col
        xs.append(h)
    return jnp.concatenate(xs, axis=-1) @ p["W"] + p["b"]


def _degree_sc(srcw, dstw, n):
    ones = jnp.zeros((_NPAD_SC, 128), jnp.float32).at[:n, :1].set(1.0)
    zeros_pad = jnp.zeros((_NPAD_SC, 128), jnp.float32)
    s = _sc_segment_sum(ones, srcw, dstw, zeros_pad, 128)
    return s[:n, 0]


def kernel(pos, x, edge_index, params):
    src, dst = edge_index[0], edge_index[1]
    n = pos.shape[0]
    x1 = _dyn_edge_conv(params["conv1"], pos, KNN_K)
    x2 = _dyn_edge_conv(params["conv2"], x1, KNN_K)
    out_d = _mlp_apply(params["lin1"], jnp.concatenate([x1, x2], axis=-1))
    srcw, dstw = _pad_edges(src, dst, n)
    deg = _degree_sc(srcw, dstw, n)
    dis = jnp.where(deg > 0, 1.0 / jnp.sqrt(jnp.maximum(deg, 1.0)), 0.0)
    g1 = jax.nn.relu(_tag_conv_sc(params["tag1"], x, srcw, dstw, dis, n))
    g2 = jax.nn.relu(_tag_conv_sc(params["tag2"], g1, srcw, dstw, dis, n))
    out_g = _mlp_apply(params["lin_g1"], jnp.concatenate([g1, g2], axis=-1))
    h = jnp.concatenate([out_d, out_g], axis=-1)
    h = _mlp_apply(params["mix1"], h)
    h = _mlp_apply(params["mix2"], h)
    return h @ params["out"]["W"] + params["out"]["b"]
